# Initial kernel scaffold; baseline (speedup 1.0000x reference)
import jax
import jax.numpy as jnp
from jax.experimental import pallas as pl


def kernel(atom_xyz, atom_types, surf_xyz, surf_curvatures, W_a0, b_a0, W_a1, b_a1, sa1_W0, sa1_b0, sa1_W1, sa1_b1, sa1_Ws, sa1_bs, sa2_W0, sa2_b0, sa2_W1, sa2_b1, sa2_Ws, sa2_bs):
    raise NotImplementedError("write your pallas kernel here")


# speedup vs baseline: 7.0264x; 7.0264x over previous
import functools

import jax
import jax.numpy as jnp
from jax import lax
from jax.experimental import pallas as pl
from jax.experimental.pallas import tpu as pltpu
from jax.experimental.pallas import tpu_sc as plsc

S = 8192
A = 4096
NSAMPLE = 16
RAD2 = 0.25
BIG = 3.0e38
KNN_ROWS = 128
TILE = 1024


def _min_and_arg(d2, ids, ncols):
    m = jnp.min(d2, axis=1, keepdims=True)
    j = jnp.min(jnp.where(d2 <= m, ids, ncols), axis=1, keepdims=True)
    return m, j


def _knn16_body(xyz_ref, xyzt_ref, idx_ref):
    x = xyz_ref[...]
    xt = xyzt_ref[...]
    dot = jnp.dot(x.astype(jnp.bfloat16), xt.astype(jnp.bfloat16),
                  preferred_element_type=jnp.float32)
    rn = jnp.sum(x * x, axis=1, keepdims=True)
    cn = jnp.sum(xt * xt, axis=0, keepdims=True)
    d2 = (rn + cn) - 2.0 * dot
    rows = x.shape[0]
    ids = lax.broadcasted_iota(jnp.int32, (rows, S), 1)
    self_id = (pl.program_id(0) * rows
               + lax.broadcasted_iota(jnp.int32, (rows, 1), 0))
    cols = lax.broadcasted_iota(jnp.int32, (rows, NSAMPLE), 1)
    out = jnp.zeros((rows, NSAMPLE), jnp.int32)
    for k in range(NSAMPLE):
        m, j = _min_and_arg(d2, ids, S)
        sel = jnp.where(m <= RAD2, j, self_id)
        out = jnp.where(cols == k, sel, out)
        if k + 1 < NSAMPLE:
            d2 = jnp.where(ids == j, BIG, d2)
    idx_ref[...] = out


def _knn3_body(xyz_ref, axt_ref, idx_ref, w_ref):
    x = xyz_ref[...]
    at = axt_ref[...]
    dot = jnp.dot(x.astype(jnp.bfloat16), at.astype(jnp.bfloat16),
                  preferred_element_type=jnp.float32)
    rn = jnp.sum(x * x, axis=1, keepdims=True)
    cn = jnp.sum(at * at, axis=0, keepdims=True)
    d2 = (rn + cn) - 2.0 * dot
    rows = x.shape[0]
    ids = lax.broadcasted_iota(jnp.int32, (rows, A), 1)
    cols = lax.broadcasted_iota(jnp.int32, (rows, 3), 1)
    idx_out = jnp.zeros((rows, 3), jnp.int32)
    w_out = jnp.zeros((rows, 3), jnp.float32)
    for k in range(3):
        m, j = _min_and_arg(d2, ids, A)
        d = jnp.sqrt(jnp.maximum(m, 1e-10))
        idx_out = jnp.where(cols == k, j, idx_out)
        w_out = jnp.where(cols == k, 1.0 / (d + 1e-8), w_out)
        if k < 2:
            d2 = jnp.where(ids == j, BIG, d2)
    w_out = w_out / jnp.sum(w_out, axis=1, keepdims=True)
    idx_ref[...] = idx_out
    w_ref[...] = w_out


def _atom_mlp_body(t_ref, w0t_ref, b0_ref, w1t_ref, b1_ref, out_ref):
    h = jnp.maximum(
        jnp.dot(t_ref[...], w0t_ref[...],
                preferred_element_type=jnp.float32) + b0_ref[...], 0.0)
    out_ref[...] = jnp.maximum(
        jnp.dot(h, w1t_ref[...],
                preferred_element_type=jnp.float32) + b1_ref[...], 0.0)


def _qproj_body(g3_ref, w_ref, curv_ref, xyz_ref,
                w0ftc_ref, w0ftq_ref, w0xt_ref, wstc_ref, wstq_ref,
                r_ref, q_ref, s_ref):
    wv = w_ref[...]
    q = (wv[:, 0:1] * g3_ref[0] + wv[:, 1:2] * g3_ref[1]
         + wv[:, 2:3] * g3_ref[2])
    x = xyz_ref[...]
    w0xt = w0xt_ref[...]
    qp = (x[:, 0:1] * w0xt[0:1, :] + x[:, 1:2] * w0xt[1:2, :]
          + x[:, 2:3] * w0xt[2:3, :])
    curv = curv_ref[...]
    p = (jnp.dot(curv, w0ftc_ref[...], preferred_element_type=jnp.float32)
         + jnp.dot(q, w0ftq_ref[...], preferred_element_type=jnp.float32))
    r_ref[...] = p + qp
    q_ref[...] = qp
    s_ref[...] = (jnp.dot(curv, wstc_ref[...],
                          preferred_element_type=jnp.float32)
                  + jnp.dot(q, wstq_ref[...],
                            preferred_element_type=jnp.float32))


def _pool_feats(g_ref, qv, w1t, b0, b1):
    pooled = None
    for k in range(NSAMPLE):
        h1 = jnp.maximum(g_ref[k] - qv + b0, 0.0)
        h2 = jnp.maximum(
            jnp.dot(h1, w1t, preferred_element_type=jnp.float32) + b1, 0.0)
        pooled = h2 if pooled is None else jnp.maximum(pooled, h2)
    return pooled


def _sa1_body(g_ref, q_ref, s_ref, xyz_ref, w1t_ref, b0_ref, b1_ref, bs_ref,
              w0ft2_ref, w0xt2_ref, wst2_ref,
              feats_ref, r2_ref, q2_ref, s2_ref):
    pooled = _pool_feats(g_ref, q_ref[...], w1t_ref[...], b0_ref[...],
                         b1_ref[...])
    f2 = jnp.maximum(pooled + s_ref[...] + bs_ref[...], 0.0)
    feats_ref[...] = f2
    x = xyz_ref[...]
    w0xt2 = w0xt2_ref[...]
    q2 = (x[:, 0:1] * w0xt2[0:1, :] + x[:, 1:2] * w0xt2[1:2, :]
          + x[:, 2:3] * w0xt2[2:3, :])
    q2_ref[...] = q2
    r2_ref[...] = jnp.dot(f2, w0ft2_ref[...],
                          preferred_element_type=jnp.float32) + q2
    s2_ref[...] = jnp.dot(f2, wst2_ref[...],
                          preferred_element_type=jnp.float32)


def _sa2_body(g_ref, q_ref, s_ref, w1t_ref, b0_ref, b1_ref, bs_ref,
              feats_ref):
    pooled = _pool_feats(g_ref, q_ref[...], w1t_ref[...], b0_ref[...],
                         b1_ref[...])
    feats_ref[...] = jnp.maximum(pooled + s_ref[...] + bs_ref[...], 0.0)


def _sc_gather(table, idx_flat):
    b_total = idx_flat.shape[0]
    info = plsc.get_sparse_core_info()
    nw = info.num_cores * info.num_subcores
    ch = 128
    bpw = b_total // nw
    nch = bpw // ch
    mesh = plsc.VectorSubcoreMesh(core_axis_name="c", subcore_axis_name="s")

    @functools.partial(
        pl.kernel, mesh=mesh,
        compiler_params=pltpu.CompilerParams(use_tc_tiling_on_sc=False),
        out_type=jax.ShapeDtypeStruct((b_total, 32), jnp.float32),
        scratch_types=[
            pltpu.VMEM((ch,), jnp.int32),
            pltpu.VMEM((ch, 32), jnp.float32),
            pltpu.SemaphoreType.DMA,
        ],
    )
    def k(table_hbm, idx_hbm, out_hbm, idx_v, rows_v, sem):
        wid = lax.axis_index("s") * info.num_cores + lax.axis_index("c")
        base = wid * bpw

        def body(c, carry):
            off = base + c * ch
            pltpu.sync_copy(idx_hbm.at[pl.ds(off, ch)], idx_v)
            pltpu.async_copy(table_hbm.at[idx_v], rows_v, sem).wait()
            pltpu.sync_copy(rows_v, out_hbm.at[pl.ds(off, ch)])
            return carry

        lax.fori_loop(0, nch, body, 0)

    return k(table, idx_flat)


def _row_spec(t, d):
    return pl.BlockSpec((t, d), lambda i: (i, 0))


def _full_spec(shape):
    nz = (0,) * len(shape)
    return pl.BlockSpec(shape, lambda i: nz)


def kernel(atom_xyz, atom_types, surf_xyz, surf_curvatures,
           W_a0, b_a0, W_a1, b_a1,
           sa1_W0, sa1_b0, sa1_W1, sa1_b1, sa1_Ws, sa1_bs,
           sa2_W0, sa2_b0, sa2_W1, sa2_b1, sa2_Ws, sa2_bs):
    xyz = surf_xyz
    f32 = jnp.float32

    idx = pl.pallas_call(
        _knn16_body,
        grid=(S // KNN_ROWS,),
        in_specs=[_row_spec(KNN_ROWS, 3), _full_spec((3, S))],
        out_specs=_row_spec(KNN_ROWS, NSAMPLE),
        out_shape=jax.ShapeDtypeStruct((S, NSAMPLE), jnp.int32),
    )(xyz, xyz.T)
    idx_flat = idx.T.reshape(-1)

    idx3, w3 = pl.pallas_call(
        _knn3_body,
        grid=(S // KNN_ROWS,),
        in_specs=[_row_spec(KNN_ROWS, 3), _full_spec((3, A))],
        out_specs=[_row_spec(KNN_ROWS, 3), _row_spec(KNN_ROWS, 3)],
        out_shape=[jax.ShapeDtypeStruct((S, 3), jnp.int32),
                   jax.ShapeDtypeStruct((S, 3), f32)],
    )(xyz, atom_xyz.T)
    idx3_flat = idx3.T.reshape(-1)

    atom_feats = pl.pallas_call(
        _atom_mlp_body,
        grid=(1,),
        in_specs=[_full_spec((A, 6)), _full_spec((6, 16)),
                  _full_spec((1, 16)), _full_spec((16, 32)),
                  _full_spec((1, 32))],
        out_specs=_full_spec((A, 32)),
        out_shape=jax.ShapeDtypeStruct((A, 32), f32),
    )(atom_types, W_a0.T, b_a0.reshape(1, 16), W_a1.T, b_a1.reshape(1, 32))

    ga = _sc_gather(atom_feats, idx3_flat).reshape(3, S, 32)
    r1, q1, s1 = pl.pallas_call(
        _qproj_body,
        grid=(S // TILE,),
        in_specs=[pl.BlockSpec((3, TILE, 32), lambda i: (0, i, 0)),
                  _row_spec(TILE, 3), _row_spec(TILE, 10), _row_spec(TILE, 3),
                  _full_spec((10, 32)), _full_spec((32, 32)),
                  _full_spec((3, 32)), _full_spec((10, 32)),
                  _full_spec((32, 32))],
        out_specs=[_row_spec(TILE, 32)] * 3,
        out_shape=[jax.ShapeDtypeStruct((S, 32), f32)] * 3,
    )(ga, w3, surf_curvatures, xyz,
      sa1_W0[:, 3:13].T, sa1_W0[:, 13:].T, sa1_W0[:, :3].T,
      sa1_Ws[:, :10].T, sa1_Ws[:, 10:].T)

    g1 = _sc_gather(r1, idx_flat).reshape(NSAMPLE, S, 32)
    feats2, r2, q2, s2 = pl.pallas_call(
        _sa1_body,
        grid=(S // TILE,),
        in_specs=[pl.BlockSpec((NSAMPLE, TILE, 32), lambda i: (0, i, 0)),
                  _row_spec(TILE, 32), _row_spec(TILE, 32),
                  _row_spec(TILE, 3),
                  _full_spec((32, 32)), _full_spec((1, 32)),
                  _full_spec((1, 32)), _full_spec((1, 32)),
                  _full_spec((32, 32)), _full_spec((3, 32)),
                  _full_spec((32, 32))],
        out_specs=[_row_spec(TILE, 32)] * 4,
        out_shape=[jax.ShapeDtypeStruct((S, 32), f32)] * 4,
    )(g1, q1, s1, xyz, sa1_W1.T, sa1_b0.reshape(1, 32),
      sa1_b1.reshape(1, 32), sa1_bs.reshape(1, 32),
      sa2_W0[:, 3:].T, sa2_W0[:, :3].T, sa2_Ws.T)

    g2 = _sc_gather(r2, idx_flat).reshape(NSAMPLE, S, 32)
    out = pl.pallas_call(
        _sa2_body,
        grid=(S // TILE,),
        in_specs=[pl.BlockSpec((NSAMPLE, TILE, 32), lambda i: (0, i, 0)),
                  _row_spec(TILE, 32), _row_spec(TILE, 32),
                  _full_spec((32, 32)), _full_spec((1, 32)),
                  _full_spec((1, 32)), _full_spec((1, 32))],
        out_specs=_row_spec(TILE, 32),
        out_shape=jax.ShapeDtypeStruct((S, 32), f32),
    )(g2, q2, s2, sa2_W1.T, sa2_b0.reshape(1, 32),
      sa2_b1.reshape(1, 32), sa2_bs.reshape(1, 32))
    return out


# speedup vs baseline: 8.3186x; 1.1839x over previous
import functools

import jax
import jax.numpy as jnp
from jax import lax
from jax.experimental import pallas as pl
from jax.experimental.pallas import tpu as pltpu
from jax.experimental.pallas import tpu_sc as plsc

S = 8192
A = 4096
NSAMPLE = 16
RAD2 = 0.25
BIG = 3.0e38
KNN_ROWS = 128
TILE = 1024


def _min_and_arg(d2, ids, ncols):
    m = jnp.min(d2, axis=1, keepdims=True)
    j = jnp.min(jnp.where(d2 <= m, ids, ncols), axis=1, keepdims=True)
    return m, j


def _knn16_body(xyz_ref, xyzt_ref, idx_ref):
    x = xyz_ref[...]
    xt = xyzt_ref[...]
    dot = jnp.dot(x.astype(jnp.bfloat16), xt.astype(jnp.bfloat16),
                  preferred_element_type=jnp.float32)
    rn = jnp.sum(x * x, axis=1, keepdims=True)
    cn = jnp.sum(xt * xt, axis=0, keepdims=True)
    d2 = (rn + cn) - 2.0 * dot
    rows = x.shape[0]
    ids = lax.broadcasted_iota(jnp.int32, (rows, S), 1).astype(jnp.float32)
    self_id = (pl.program_id(0) * rows
               + lax.broadcasted_iota(jnp.int32, (rows, 1), 0)).astype(
                   jnp.float32)
    cols = lax.broadcasted_iota(jnp.int32, (rows, NSAMPLE), 1)
    out = jnp.zeros((rows, NSAMPLE), jnp.float32)
    for k in range(NSAMPLE):
        m, j = _min_and_arg(d2, ids, float(S))
        sel = jnp.where(m <= RAD2, j, self_id)
        out = jnp.where(cols == k, sel, out)
        if k + 1 < NSAMPLE:
            d2 = jnp.where(ids == j, BIG, d2)
    idx_ref[...] = out.astype(jnp.int32).T


def _knn3_body(xyz_ref, axt_ref, idx_ref, w_ref):
    x = xyz_ref[...]
    at = axt_ref[...]
    dot = jnp.dot(x.astype(jnp.bfloat16), at.astype(jnp.bfloat16),
                  preferred_element_type=jnp.float32)
    rn = jnp.sum(x * x, axis=1, keepdims=True)
    cn = jnp.sum(at * at, axis=0, keepdims=True)
    d2 = (rn + cn) - 2.0 * dot
    rows = x.shape[0]
    ids = lax.broadcasted_iota(jnp.int32, (rows, A), 1).astype(jnp.float32)
    cols = lax.broadcasted_iota(jnp.int32, (rows, 3), 1)
    idx_out = jnp.zeros((rows, 3), jnp.float32)
    w_out = jnp.zeros((rows, 3), jnp.float32)
    for k in range(3):
        m, j = _min_and_arg(d2, ids, float(A))
        d = jnp.sqrt(jnp.maximum(m, 1e-10))
        idx_out = jnp.where(cols == k, j, idx_out)
        w_out = jnp.where(cols == k, 1.0 / (d + 1e-8), w_out)
        if k < 2:
            d2 = jnp.where(ids == j, BIG, d2)
    w_out = w_out / jnp.sum(w_out, axis=1, keepdims=True)
    idx_ref[...] = idx_out.astype(jnp.int32).T
    w_ref[...] = w_out


def _atom_mlp_body(t_ref, w0t_ref, b0_ref, w1t_ref, b1_ref, out_ref):
    h = jnp.maximum(
        jnp.dot(t_ref[...], w0t_ref[...],
                preferred_element_type=jnp.float32) + b0_ref[...], 0.0)
    out_ref[...] = jnp.maximum(
        jnp.dot(h, w1t_ref[...],
                preferred_element_type=jnp.float32) + b1_ref[...], 0.0)


def _qproj_body(g3_ref, w_ref, curv_ref, xyz_ref,
                w0ftc_ref, w0ftq_ref, w0xt_ref, wstc_ref, wstq_ref,
                r_ref, q_ref, s_ref):
    wv = w_ref[...]
    q = (wv[:, 0:1] * g3_ref[0] + wv[:, 1:2] * g3_ref[1]
         + wv[:, 2:3] * g3_ref[2])
    x = xyz_ref[...]
    w0xt = w0xt_ref[...]
    qp = (x[:, 0:1] * w0xt[0:1, :] + x[:, 1:2] * w0xt[1:2, :]
          + x[:, 2:3] * w0xt[2:3, :])
    curv = curv_ref[...]
    p = (jnp.dot(curv, w0ftc_ref[...], preferred_element_type=jnp.float32)
         + jnp.dot(q, w0ftq_ref[...], preferred_element_type=jnp.float32))
    r_ref[...] = p + qp
    q_ref[...] = qp
    s_ref[...] = (jnp.dot(curv, wstc_ref[...],
                          preferred_element_type=jnp.float32)
                  + jnp.dot(q, wstq_ref[...],
                            preferred_element_type=jnp.float32))


def _pool_feats(g_ref, qv, w1t, b0, b1):
    pooled = None
    for k in range(NSAMPLE):
        h1 = jnp.maximum(g_ref[k] - qv + b0, 0.0)
        h2 = jnp.maximum(
            jnp.dot(h1, w1t, preferred_element_type=jnp.float32) + b1, 0.0)
        pooled = h2 if pooled is None else jnp.maximum(pooled, h2)
    return pooled


def _sa1_body(g_ref, q_ref, s_ref, xyz_ref, w1t_ref, b0_ref, b1_ref, bs_ref,
              w0ft2_ref, w0xt2_ref, wst2_ref,
              feats_ref, r2_ref, q2_ref, s2_ref):
    pooled = _pool_feats(g_ref, q_ref[...], w1t_ref[...], b0_ref[...],
                         b1_ref[...])
    f2 = jnp.maximum(pooled + s_ref[...] + bs_ref[...], 0.0)
    feats_ref[...] = f2
    x = xyz_ref[...]
    w0xt2 = w0xt2_ref[...]
    q2 = (x[:, 0:1] * w0xt2[0:1, :] + x[:, 1:2] * w0xt2[1:2, :]
          + x[:, 2:3] * w0xt2[2:3, :])
    q2_ref[...] = q2
    r2_ref[...] = jnp.dot(f2, w0ft2_ref[...],
                          preferred_element_type=jnp.float32) + q2
    s2_ref[...] = jnp.dot(f2, wst2_ref[...],
                          preferred_element_type=jnp.float32)


def _sa2_body(g_ref, q_ref, s_ref, w1t_ref, b0_ref, b1_ref, bs_ref,
              feats_ref):
    pooled = _pool_feats(g_ref, q_ref[...], w1t_ref[...], b0_ref[...],
                         b1_ref[...])
    feats_ref[...] = jnp.maximum(pooled + s_ref[...] + bs_ref[...], 0.0)


def _sc_gather(table, idx_flat):
    b_total = idx_flat.shape[0]
    info = plsc.get_sparse_core_info()
    nw = info.num_cores * info.num_subcores
    ch = 128
    bpw = b_total // nw
    nch = bpw // ch
    mesh = plsc.VectorSubcoreMesh(core_axis_name="c", subcore_axis_name="s")

    @functools.partial(
        pl.kernel, mesh=mesh,
        compiler_params=pltpu.CompilerParams(use_tc_tiling_on_sc=False),
        out_type=jax.ShapeDtypeStruct((b_total, 32), jnp.float32),
        scratch_types=[
            pltpu.VMEM((ch,), jnp.int32),
            pltpu.VMEM((ch, 32), jnp.float32),
            pltpu.SemaphoreType.DMA,
        ],
    )
    def k(table_hbm, idx_hbm, out_hbm, idx_v, rows_v, sem):
        wid = lax.axis_index("s") * info.num_cores + lax.axis_index("c")
        base = wid * bpw

        def body(c, carry):
            off = base + c * ch
            pltpu.sync_copy(idx_hbm.at[pl.ds(off, ch)], idx_v)
            pltpu.async_copy(table_hbm.at[idx_v], rows_v, sem).wait()
            pltpu.sync_copy(rows_v, out_hbm.at[pl.ds(off, ch)])
            return carry

        lax.fori_loop(0, nch, body, 0)

    return k(table, idx_flat)


def _row_spec(t, d):
    return pl.BlockSpec((t, d), lambda i: (i, 0))


def _full_spec(shape):
    nz = (0,) * len(shape)
    return pl.BlockSpec(shape, lambda i: nz)


def kernel(atom_xyz, atom_types, surf_xyz, surf_curvatures,
           W_a0, b_a0, W_a1, b_a1,
           sa1_W0, sa1_b0, sa1_W1, sa1_b1, sa1_Ws, sa1_bs,
           sa2_W0, sa2_b0, sa2_W1, sa2_b1, sa2_Ws, sa2_bs):
    xyz = surf_xyz
    f32 = jnp.float32

    idx3, w3 = pl.pallas_call(
        _knn3_body,
        grid=(S // KNN_ROWS,),
        in_specs=[_row_spec(KNN_ROWS, 3), _full_spec((3, A))],
        out_specs=[pl.BlockSpec((3, KNN_ROWS), lambda i: (0, i)),
                   _row_spec(KNN_ROWS, 3)],
        out_shape=[jax.ShapeDtypeStruct((3, S), jnp.int32),
                   jax.ShapeDtypeStruct((S, 3), f32)],
    )(xyz, atom_xyz.T)
    idx3_flat = idx3.reshape(-1)

    atom_feats = pl.pallas_call(
        _atom_mlp_body,
        grid=(1,),
        in_specs=[_full_spec((A, 6)), _full_spec((6, 16)),
                  _full_spec((1, 16)), _full_spec((16, 32)),
                  _full_spec((1, 32))],
        out_specs=_full_spec((A, 32)),
        out_shape=jax.ShapeDtypeStruct((A, 32), f32),
    )(atom_types, W_a0.T, b_a0.reshape(1, 16), W_a1.T, b_a1.reshape(1, 32))

    ga = _sc_gather(atom_feats, idx3_flat).reshape(3, S, 32)

    idx = pl.pallas_call(
        _knn16_body,
        grid=(S // KNN_ROWS,),
        in_specs=[_row_spec(KNN_ROWS, 3), _full_spec((3, S))],
        out_specs=pl.BlockSpec((NSAMPLE, KNN_ROWS), lambda i: (0, i)),
        out_shape=jax.ShapeDtypeStruct((NSAMPLE, S), jnp.int32),
    )(xyz, xyz.T)
    idx_flat = idx.reshape(-1)
    r1, q1, s1 = pl.pallas_call(
        _qproj_body,
        grid=(S // TILE,),
        in_specs=[pl.BlockSpec((3, TILE, 32), lambda i: (0, i, 0)),
                  _row_spec(TILE, 3), _row_spec(TILE, 10), _row_spec(TILE, 3),
                  _full_spec((10, 32)), _full_spec((32, 32)),
                  _full_spec((3, 32)), _full_spec((10, 32)),
                  _full_spec((32, 32))],
        out_specs=[_row_spec(TILE, 32)] * 3,
        out_shape=[jax.ShapeDtypeStruct((S, 32), f32)] * 3,
    )(ga, w3, surf_curvatures, xyz,
      sa1_W0[:, 3:13].T, sa1_W0[:, 13:].T, sa1_W0[:, :3].T,
      sa1_Ws[:, :10].T, sa1_Ws[:, 10:].T)

    g1 = _sc_gather(r1, idx_flat).reshape(NSAMPLE, S, 32)
    feats2, r2, q2, s2 = pl.pallas_call(
        _sa1_body,
        grid=(S // TILE,),
        in_specs=[pl.BlockSpec((NSAMPLE, TILE, 32), lambda i: (0, i, 0)),
                  _row_spec(TILE, 32), _row_spec(TILE, 32),
                  _row_spec(TILE, 3),
                  _full_spec((32, 32)), _full_spec((1, 32)),
                  _full_spec((1, 32)), _full_spec((1, 32)),
                  _full_spec((32, 32)), _full_spec((3, 32)),
                  _full_spec((32, 32))],
        out_specs=[_row_spec(TILE, 32)] * 4,
        out_shape=[jax.ShapeDtypeStruct((S, 32), f32)] * 4,
    )(g1, q1, s1, xyz, sa1_W1.T, sa1_b0.reshape(1, 32),
      sa1_b1.reshape(1, 32), sa1_bs.reshape(1, 32),
      sa2_W0[:, 3:].T, sa2_W0[:, :3].T, sa2_Ws.T)

    g2 = _sc_gather(r2, idx_flat).reshape(NSAMPLE, S, 32)
    out = pl.pallas_call(
        _sa2_body,
        grid=(S // TILE,),
        in_specs=[pl.BlockSpec((NSAMPLE, TILE, 32), lambda i: (0, i, 0)),
                  _row_spec(TILE, 32), _row_spec(TILE, 32),
                  _full_spec((32, 32)), _full_spec((1, 32)),
                  _full_spec((1, 32)), _full_spec((1, 32))],
        out_specs=_row_spec(TILE, 32),
        out_shape=jax.ShapeDtypeStruct((S, 32), f32),
    )(g2, q2, s2, sa2_W1.T, sa2_b0.reshape(1, 32),
      sa2_b1.reshape(1, 32), sa2_bs.reshape(1, 32))
    return out
